# baseline (speedup 1.0000x reference)
import functools

import jax
import jax.numpy as jnp
from jax import lax
from jax.experimental import pallas as pl
from jax.experimental.pallas import tpu as pltpu
from jax.experimental.pallas import tpu_sc as plsc

VOCAB_P1 = 1000001
EMBED_DIM = 32
BATCH = 16384
_LANES = 128

_info = plsc.get_sparse_core_info()
_NC, _NS = _info.num_cores, _info.num_subcores
_NW = _NC * _NS
_B_PER_W = BATCH // _NW
_WAVE = 8
_N_WAVES = _B_PER_W // _WAVE


def _gather_body(idx_hbm, tab_hbm, out_hbm, idx_v, slab_a, slab_b, out_v,
                 sem_a, sem_b):
    wid = lax.axis_index("s") * _NC + lax.axis_index("c")
    base = wid * _B_PER_W
    pltpu.sync_copy(idx_hbm.at[pl.ds(base, _B_PER_W)], idx_v)
    c_lo = lax.iota(jnp.int32, 16)
    c_hi = c_lo + 16

    def load16(g):
        return idx_v[pl.ds(g * 2 * _WAVE, 16)]

    def fire(vec16, half, slab, sem):
        for k in range(_WAVE):
            r = vec16[half * _WAVE + k]
            blk = pl.multiple_of(r & ~(_LANES - 1), _LANES)
            pltpu.async_copy(
                tab_hbm.at[:, pl.ds(blk, _LANES)], slab.at[k], sem
            )

    def drain_extract(vec16, half, w, slab, sem):
        for k in range(_WAVE):
            pltpu.make_async_copy(
                tab_hbm.at[:, pl.ds(0, _LANES)], slab.at[k], sem
            ).wait()
        lane = vec16 & (_LANES - 1)
        for k in range(_WAVE):
            l_vec = jnp.full((16,), lane[half * _WAVE + k], dtype=jnp.int32)
            lo = plsc.load_gather(slab.at[k], [c_lo, l_vec])
            hi = plsc.load_gather(slab.at[k], [c_hi, l_vec])
            j_vec = jnp.full((16,), w * _WAVE + k, dtype=jnp.int32)
            plsc.store_scatter(out_v, [c_lo, j_vec], lo)
            plsc.store_scatter(out_v, [c_hi, j_vec], hi)

    fire(load16(0), 0, slab_a, sem_a)

    def step(g, carry):
        vec16 = load16(g)
        w0 = 2 * g
        fire(vec16, 1, slab_b, sem_b)
        drain_extract(vec16, 0, w0, slab_a, sem_a)

        @pl.when(g + 1 < _N_WAVES // 2)
        def _():
            fire(load16(g + 1), 0, slab_a, sem_a)

        drain_extract(vec16, 1, w0 + 1, slab_b, sem_b)
        return carry

    lax.fori_loop(0, _N_WAVES // 2, step, None)
    pltpu.sync_copy(out_v, out_hbm.at[:, pl.ds(base, _B_PER_W)])


@jax.jit
def kernel(item_id, id_table):
    mesh = plsc.VectorSubcoreMesh(core_axis_name="c", subcore_axis_name="s")
    gather = functools.partial(
        pl.kernel,
        mesh=mesh,
        out_type=jax.ShapeDtypeStruct((EMBED_DIM, BATCH), jnp.float32),
        scratch_types=[
            pltpu.VMEM((_B_PER_W,), jnp.int32),
            pltpu.VMEM((_WAVE, EMBED_DIM, _LANES), jnp.float32),
            pltpu.VMEM((_WAVE, EMBED_DIM, _LANES), jnp.float32),
            pltpu.VMEM((EMBED_DIM, _B_PER_W), jnp.float32),
            pltpu.SemaphoreType.DMA,
            pltpu.SemaphoreType.DMA,
        ],
        compiler_params=pltpu.CompilerParams(needs_layout_passes=False),
    )(_gather_body)
    out_t = gather(item_id.astype(jnp.int32), id_table.T)
    return out_t.T


# speedup vs baseline: 1.0125x; 1.0125x over previous
import functools

import jax
import jax.numpy as jnp
from jax import lax
from jax.experimental import pallas as pl
from jax.experimental.pallas import tpu as pltpu
from jax.experimental.pallas import tpu_sc as plsc

VOCAB_P1 = 1000001
EMBED_DIM = 32
BATCH = 16384
_LANES = 128

_info = plsc.get_sparse_core_info()
_NC, _NS = _info.num_cores, _info.num_subcores
_NW = _NC * _NS
_B_PER_W = BATCH // _NW
_WAVE = 16
_N_WAVES = _B_PER_W // _WAVE


def _gather_body(idx_hbm, tab_hbm, out_hbm, idx_v, slab_v, out_v, sem):
    wid = lax.axis_index("s") * _NC + lax.axis_index("c")
    base = wid * _B_PER_W
    pltpu.sync_copy(idx_hbm.at[pl.ds(base, _B_PER_W)], idx_v)
    c_lo = lax.iota(jnp.int32, 16)
    c_hi = c_lo + 16

    def wave(g, carry):
        vec = idx_v[pl.ds(g * _WAVE, _WAVE)]
        copies = []
        for k in range(_WAVE):
            r = vec[k]
            blk = pl.multiple_of(r & ~(_LANES - 1), _LANES)
            copies.append(
                pltpu.async_copy(
                    tab_hbm.at[:, pl.ds(blk, _LANES)], slab_v.at[k], sem
                )
            )
        lane = vec & (_LANES - 1)
        for k in range(_WAVE):
            copies[k].wait()
            l_vec = jnp.full((16,), lane[k], dtype=jnp.int32)
            lo = plsc.load_gather(slab_v.at[k], [c_lo, l_vec])
            hi = plsc.load_gather(slab_v.at[k], [c_hi, l_vec])
            j_vec = jnp.full((16,), g * _WAVE + k, dtype=jnp.int32)
            plsc.store_scatter(out_v, [c_lo, j_vec], lo)
            plsc.store_scatter(out_v, [c_hi, j_vec], hi)
        return carry

    lax.fori_loop(0, _N_WAVES, wave, None)
    pltpu.sync_copy(out_v, out_hbm.at[:, pl.ds(base, _B_PER_W)])


@jax.jit
def kernel(item_id, id_table):
    mesh = plsc.VectorSubcoreMesh(core_axis_name="c", subcore_axis_name="s")
    gather = functools.partial(
        pl.kernel,
        mesh=mesh,
        out_type=jax.ShapeDtypeStruct((EMBED_DIM, BATCH), jnp.float32),
        scratch_types=[
            pltpu.VMEM((_B_PER_W,), jnp.int32),
            pltpu.VMEM((_WAVE, EMBED_DIM, _LANES), jnp.float32),
            pltpu.VMEM((EMBED_DIM, _B_PER_W), jnp.float32),
            pltpu.SemaphoreType.DMA,
        ],
        compiler_params=pltpu.CompilerParams(needs_layout_passes=False),
    )(_gather_body)
    out_t = gather(item_id.astype(jnp.int32), id_table.T)
    return out_t.T


# speedup vs baseline: 1.0997x; 1.0862x over previous
import functools

import jax
import jax.numpy as jnp
from jax import lax
from jax.experimental import pallas as pl
from jax.experimental.pallas import tpu as pltpu
from jax.experimental.pallas import tpu_sc as plsc

VOCAB_P1 = 1000001
VOCAB_PAD = 1000064
NTILE = VOCAB_PAD // 128
EMBED_DIM = 32
BATCH = 16384

_info = plsc.get_sparse_core_info()
_NC, _NS = _info.num_cores, _info.num_subcores
_NW = _NC * _NS
_B_PER_W = BATCH // _NW


def _gather_body(idx_hbm, tabf_hbm, out_hbm, idx_v, flat_v, rows_v, out_v, sem):
    wid = lax.axis_index("s") * _NC + lax.axis_index("c")
    base = wid * _B_PER_W
    pltpu.sync_copy(idx_hbm.at[pl.ds(base, _B_PER_W)], idx_v)

    def build(i, carry):
        r = idx_v[pl.ds(i * 16, 16)]
        hi = (r >> 7) * 1024 + (r & 127)
        for c in range(EMBED_DIM):
            off = (c >> 3) * (NTILE * 1024) + (c & 7) * 128
            flat_v[pl.ds(c * _B_PER_W + i * 16, 16)] = hi + off
        return carry

    lax.fori_loop(0, _B_PER_W // 16, build, None)

    copies = []
    for c in range(EMBED_DIM):
        copies.append(
            pltpu.async_copy(
                tabf_hbm.at[flat_v.at[pl.ds(c * _B_PER_W, _B_PER_W)]],
                rows_v.at[pl.ds(c * _B_PER_W, _B_PER_W)],
                sem,
            )
        )
    for cp in copies:
        cp.wait()

    c_vec = lax.iota(jnp.int32, 16)

    def regroup(i, carry):
        for c2 in range(2):
            v = rows_v[pl.ds((i * 2 + c2) * 16, 16)]
            cc = i * 2 + c2
            c_idx = jnp.full((16,), cc // 32, dtype=jnp.int32)
            j_vec = jnp.full((16,), (cc % 32) * 16, dtype=jnp.int32) + c_vec
            plsc.store_scatter(out_v, [c_idx, j_vec], v)
        return carry

    lax.fori_loop(0, EMBED_DIM * _B_PER_W // 32, regroup, None)
    pltpu.sync_copy(out_v, out_hbm.at[:, pl.ds(base, _B_PER_W)])


@jax.jit
def kernel(item_id, id_table):
    tab_pad = jnp.pad(id_table, ((0, VOCAB_PAD - VOCAB_P1), (0, 0)))
    v2 = tab_pad.T.reshape(4, 8, NTILE, 128)
    v = v2.transpose(0, 2, 1, 3)
    tab_flat = v.reshape(-1)
    mesh = plsc.VectorSubcoreMesh(core_axis_name="c", subcore_axis_name="s")
    gather = functools.partial(
        pl.kernel,
        mesh=mesh,
        out_type=jax.ShapeDtypeStruct((EMBED_DIM, BATCH), jnp.float32),
        scratch_types=[
            pltpu.VMEM((_B_PER_W,), jnp.int32),
            pltpu.VMEM((EMBED_DIM * _B_PER_W,), jnp.int32),
            pltpu.VMEM((EMBED_DIM * _B_PER_W,), jnp.float32),
            pltpu.VMEM((EMBED_DIM, _B_PER_W), jnp.float32),
            pltpu.SemaphoreType.DMA,
        ],
        compiler_params=pltpu.CompilerParams(
            use_tc_tiling_on_sc=False, needs_layout_passes=False
        ),
    )(_gather_body)
    out_t = gather(item_id.astype(jnp.int32), tab_flat)
    return out_t.T
